# baseline (speedup 1.0000x reference)
import functools

import jax
import jax.numpy as jnp
from jax import lax
from jax.experimental import pallas as pl
from jax.experimental.pallas import tpu as pltpu
from jax.experimental.pallas import tpu_sc as plsc

NC, NS = 2, 16
NW = NC * NS

CHUNK = 800


def kernel(x, table):
    B, L = x.shape
    V, D = table.shape
    N = B * L
    assert N % NW == 0
    pw = N // NW
    assert pw % CHUNK == 0
    g_steps = pw // CHUNK

    mesh = plsc.VectorSubcoreMesh(
        core_axis_name="c", subcore_axis_name="s",
        num_cores=NC, num_subcores=NS,
    )

    @functools.partial(
        pl.kernel,
        out_type=jax.ShapeDtypeStruct((N, D), jnp.float32),
        mesh=mesh,
        scratch_types=[
            pltpu.VMEM((CHUNK,), jnp.int32),
            pltpu.VMEM((CHUNK, D), jnp.float32),
            pltpu.SemaphoreType.DMA,
        ],
        compiler_params=pltpu.CompilerParams(use_tc_tiling_on_sc=False),
    )
    def emb(idx_hbm, tab_hbm, out_hbm, idx_v, rows_v, sem):
        wid = lax.axis_index("s") * NC + lax.axis_index("c")
        base = wid * pw

        def body(g, carry):
            row0 = base + g * CHUNK
            pltpu.sync_copy(idx_hbm.at[pl.ds(row0, CHUNK)], idx_v)
            pltpu.async_copy(tab_hbm.at[idx_v], rows_v, sem).wait()
            pltpu.sync_copy(rows_v, out_hbm.at[pl.ds(row0, CHUNK)])
            return carry

        lax.fori_loop(0, g_steps, body, 0)

    out = emb(x.reshape(N), table)
    return out.reshape(B, L, D)


# speedup vs baseline: 1.0458x; 1.0458x over previous
import functools

import jax
import jax.numpy as jnp
from jax import lax
from jax.experimental import pallas as pl
from jax.experimental.pallas import tpu as pltpu
from jax.experimental.pallas import tpu_sc as plsc

NC, NS = 2, 16
NW = NC * NS

CHUNK = 800
NBUF = 2


def kernel(x, table):
    B, L = x.shape
    V, D = table.shape
    N = B * L
    assert N % NW == 0
    pw = N // NW
    assert pw % CHUNK == 0
    g_steps = pw // CHUNK
    assert g_steps % 2 == 0 and g_steps >= 4

    mesh = plsc.VectorSubcoreMesh(
        core_axis_name="c", subcore_axis_name="s",
        num_cores=NC, num_subcores=NS,
    )

    scratch = (
        [pltpu.VMEM((CHUNK,), jnp.int32) for _ in range(NBUF)]
        + [pltpu.VMEM((CHUNK, D), jnp.float32) for _ in range(NBUF)]
        + [pltpu.SemaphoreType.DMA for _ in range(3 * NBUF)]
    )

    @functools.partial(
        pl.kernel,
        out_type=jax.ShapeDtypeStruct((N, D), jnp.float32),
        mesh=mesh,
        scratch_types=scratch,
        compiler_params=pltpu.CompilerParams(use_tc_tiling_on_sc=False),
    )
    def emb(idx_hbm, tab_hbm, out_hbm,
            i0, i1, r0, r1, si0, si1, sg0, sg1, so0, so1):
        idx_v = [i0, i1]
        rows_v = [r0, r1]
        i_sem = [si0, si1]
        g_sem = [sg0, sg1]
        o_sem = [so0, so1]

        wid = lax.axis_index("s") * NC + lax.axis_index("c")
        base = wid * pw

        def idx_cp(g, b):
            return pltpu.make_async_copy(
                idx_hbm.at[pl.ds(base + g * CHUNK, CHUNK)], idx_v[b], i_sem[b])

        def gat_cp(b):
            return pltpu.make_async_copy(tab_hbm.at[idx_v[b]], rows_v[b], g_sem[b])

        def out_cp(g, b):
            return pltpu.make_async_copy(
                rows_v[b], out_hbm.at[pl.ds(base + g * CHUNK, CHUNK)], o_sem[b])

        for b in range(NBUF):
            idx_cp(b, b).start()
        for b in range(NBUF):
            idx_cp(b, b).wait()
            gat_cp(b).start()
            gat_cp(b).wait()
            out_cp(b, b).start()
            idx_cp(b + NBUF, b).start()

        def pair(blk, carry):
            for b in range(NBUF):
                g = blk * NBUF + b
                idx_cp(g, b).wait()
                out_cp(g - NBUF, b).wait()
                gat_cp(b).start()
                gat_cp(b).wait()
                out_cp(g, b).start()
                idx_cp(g + NBUF, b).start()
            return carry

        lax.fori_loop(1, g_steps // NBUF - 1, pair, 0)

        for b in range(NBUF):
            g = g_steps - NBUF + b
            idx_cp(g, b).wait()
            out_cp(g - NBUF, b).wait()
            gat_cp(b).start()
            gat_cp(b).wait()
            out_cp(g, b).start()
        for b in range(NBUF):
            out_cp(g_steps - NBUF + b, b).wait()

    out = emb(x.reshape(N), table)
    return out.reshape(B, L, D)
